# Initial kernel scaffold; baseline (speedup 1.0000x reference)
import jax
import jax.numpy as jnp
from jax.experimental import pallas as pl


def kernel(positions, cos_cache, sin_cache):
    raise NotImplementedError("write your pallas kernel here")


# speedup vs baseline: 3.3409x; 3.3409x over previous
import functools

import jax
import jax.numpy as jnp
from jax import lax
from jax.experimental import pallas as pl
from jax.experimental.pallas import tpu as pltpu
from jax.experimental.pallas import tpu_sc as plsc

SEQ = 32768
DIM_HALF = 64

_info = plsc.get_sparse_core_info()
_NC, _NS = _info.num_cores, _info.num_subcores
_NW = _NC * _NS
_BPW = SEQ // _NW
_CHUNK = 512


def _make_kernel():
  mesh = plsc.VectorSubcoreMesh(core_axis_name="c", subcore_axis_name="s")

  @functools.partial(
      pl.kernel,
      mesh=mesh,
      compiler_params=pltpu.CompilerParams(use_tc_tiling_on_sc=False),
      out_type=(
          jax.ShapeDtypeStruct((SEQ, DIM_HALF), jnp.float32),
          jax.ShapeDtypeStruct((SEQ, DIM_HALF), jnp.float32),
      ),
      scratch_types=[
          pltpu.VMEM((_BPW,), jnp.int32),
          pltpu.VMEM((_CHUNK, DIM_HALF), jnp.float32),
          pltpu.VMEM((_CHUNK, DIM_HALF), jnp.float32),
          pltpu.SemaphoreType.DMA,
      ],
  )
  def rope_gather(pos_hbm, cos_hbm, sin_hbm, cos_out, sin_out,
                  idx_v, cos_v, sin_v, sem):
    wid = lax.axis_index("s") * _NC + lax.axis_index("c")
    base = wid * _BPW
    pltpu.sync_copy(pos_hbm.at[pl.ds(base, _BPW)], idx_v)
    for c in range(_BPW // _CHUNK):
      off = c * _CHUNK
      idx_c = idx_v.at[pl.ds(off, _CHUNK)]
      cp_cos = pltpu.async_copy(cos_hbm.at[idx_c], cos_v, sem)
      cp_sin = pltpu.async_copy(sin_hbm.at[idx_c], sin_v, sem)
      cp_cos.wait()
      pltpu.sync_copy(cos_v, cos_out.at[pl.ds(base + off, _CHUNK)])
      cp_sin.wait()
      pltpu.sync_copy(sin_v, sin_out.at[pl.ds(base + off, _CHUNK)])

  return rope_gather


_rope_gather = _make_kernel()


@jax.jit
def kernel(positions, cos_cache, sin_cache):
  return _rope_gather(positions.astype(jnp.int32), cos_cache, sin_cache)
